# baseline (speedup 1.0000x reference)
import jax
import jax.numpy as jnp
from jax import lax
from jax.experimental import pallas as pl
from jax.experimental.pallas import tpu as pltpu
from jax.experimental.pallas import tpu_sc as plsc

B = 16384
EMB = 64
NC = 2
NS = 16
NW = NC * NS
CHUNK = 128
CHUNKS = B // NW // CHUNK
PER_W = CHUNKS * CHUNK
LANES = 16


def _sbr_body(uid_hbm, iid_hbm, urow_hbm, irow_hbm, wT_hbm, ue_hbm, ub_hbm,
              ie_hbm, ib_hbm, se_hbm, mean_hbm, out_hbm,
              uidx_v, iidx_v, urow_v, irow_v, bu_v, bi_v, mean_v,
              U_v, I_v, S_v, W_v, out_v, sem):
    wid = lax.axis_index("s") * NC + lax.axis_index("c")
    base = wid * PER_W

    pltpu.sync_copy(uid_hbm.at[pl.ds(base, PER_W)], uidx_v)
    pltpu.sync_copy(iid_hbm.at[pl.ds(base, PER_W)], iidx_v)
    pltpu.sync_copy(mean_hbm, mean_v)
    for c in range(CHUNKS):
        pltpu.sync_copy(urow_hbm.at[pl.ds(base + c * CHUNK, CHUNK)],
                        urow_v.at[c])
        pltpu.sync_copy(irow_hbm.at[pl.ds(base + c * CHUNK, CHUNK)],
                        irow_v.at[c])

    bias_cps = []
    for c in range(CHUNKS):
        bias_cps.append(pltpu.make_async_copy(
            ub_hbm.at[uidx_v.at[pl.ds(c * CHUNK, CHUNK)]],
            bu_v.at[pl.ds(c * CHUNK, CHUNK)], sem))
        bias_cps.append(pltpu.make_async_copy(
            ib_hbm.at[iidx_v.at[pl.ds(c * CHUNK, CHUNK)]],
            bi_v.at[pl.ds(c * CHUNK, CHUNK)], sem))
    for cp in bias_cps:
        cp.start()
    for cp in bias_cps:
        cp.wait()

    lane_iota = lax.iota(jnp.int32, LANES)
    mean_vec = mean_v[...]

    for c in range(CHUNKS):
        row0 = base + c * CHUNK
        cps = [
            pltpu.make_async_copy(ue_hbm.at[urow_v.at[c]], U_v, sem),
            pltpu.make_async_copy(ie_hbm.at[irow_v.at[c]], I_v, sem),
            pltpu.make_async_copy(se_hbm.at[irow_v.at[c]], S_v, sem),
            pltpu.make_async_copy(wT_hbm.at[:, pl.ds(row0, CHUNK)], W_v, sem),
        ]
        for cp in cps:
            cp.start()
        for cp in cps:
            cp.wait()

        def group_body(g, _, c=c):
            rows = g * LANES + lane_iota
            uvec = uidx_v[pl.ds(c * CHUNK + g * LANES, LANES)]
            ivec = iidx_v[pl.ds(c * CHUNK + g * LANES, LANES)]
            uoff = lax.shift_left((uvec & 1), 6)
            ioff = lax.shift_left((ivec & 1), 6)
            acc = mean_vec
            for d in range(EMB):
                cu = plsc.load_gather(U_v, [rows, uoff + d])
                ci = plsc.load_gather(I_v, [rows, ioff + d])
                cs = plsc.load_gather(S_v, [rows, ioff + d])
                cw = W_v[d, pl.ds(g * LANES, LANES)]
                acc = acc + cu * ci + cs * cw
            acc = acc + bu_v[pl.ds(c * CHUNK + g * LANES, LANES)]
            acc = acc + bi_v[pl.ds(c * CHUNK + g * LANES, LANES)]
            out_v[pl.ds(c * CHUNK + g * LANES, LANES)] = acc
            return 0

        lax.fori_loop(0, CHUNK // LANES, group_body, 0)

    pltpu.sync_copy(out_v, out_hbm.at[pl.ds(base, PER_W)])


def kernel(u_id, i_id, UserShadow, user_emb, user_bias, item_emb, item_bias,
           shadow_i_emb, mean):
    urow = lax.shift_right_logical(u_id, 1)
    irow = lax.shift_right_logical(i_id, 1)
    ue2 = user_emb.reshape(-1, 2 * EMB)
    ie2 = item_emb.reshape(-1, 2 * EMB)
    se2 = shadow_i_emb.reshape(-1, 2 * EMB)
    wT = UserShadow.T
    ub_flat = user_bias.reshape(-1)
    ib_flat = item_bias.reshape(-1)
    mean16 = jnp.broadcast_to(mean, (LANES,))

    mesh = plsc.VectorSubcoreMesh(core_axis_name="c", subcore_axis_name="s")
    run = pl.kernel(
        _sbr_body,
        out_type=jax.ShapeDtypeStruct((B,), jnp.float32),
        mesh=mesh,
        compiler_params=pltpu.CompilerParams(
            needs_layout_passes=False, use_tc_tiling_on_sc=True),
        scratch_types=[
            pltpu.VMEM((PER_W,), jnp.int32),
            pltpu.VMEM((PER_W,), jnp.int32),
            pltpu.VMEM((CHUNKS, CHUNK), jnp.int32),
            pltpu.VMEM((CHUNKS, CHUNK), jnp.int32),
            pltpu.VMEM((PER_W,), jnp.float32),
            pltpu.VMEM((PER_W,), jnp.float32),
            pltpu.VMEM((LANES,), jnp.float32),
            pltpu.VMEM((CHUNK, 2 * EMB), jnp.float32),
            pltpu.VMEM((CHUNK, 2 * EMB), jnp.float32),
            pltpu.VMEM((CHUNK, 2 * EMB), jnp.float32),
            pltpu.VMEM((EMB, CHUNK), jnp.float32),
            pltpu.VMEM((PER_W,), jnp.float32),
            pltpu.SemaphoreType.DMA,
        ],
    )
    return run(u_id, i_id, urow, irow, wT, ue2, ub_flat, ie2, ib_flat, se2,
               mean16)
